# Initial kernel scaffold; baseline (speedup 1.0000x reference)
import jax
import jax.numpy as jnp
from jax.experimental import pallas as pl


def kernel(x, y):
    raise NotImplementedError("write your pallas kernel here")


# speedup vs baseline: 1.1105x; 1.1105x over previous
import jax
import jax.numpy as jnp
from jax.experimental import pallas as pl

_B, _H, _W, _C = 32, 16, 16, 64
_N = _H * _W
_R = _B * _N
_E = _N - 1
_ROWS = 1024
_GRID = _R // _ROWS


def _build_kernel(x_ref, edges_ref, nodes_ref, batch_ref):
    base = pl.program_id(0) * _ROWS
    r = base + jax.lax.broadcasted_iota(jnp.int32, (_ROWS, _E), 0)
    j = jax.lax.broadcasted_iota(jnp.int32, (_ROWS, _E), 1)
    i = jnp.bitwise_and(r, _N - 1)
    edges_ref[0] = r
    edges_ref[1] = r - i + j + (j >= i).astype(jnp.int32)

    rcol = base + jax.lax.broadcasted_iota(jnp.int32, (_ROWS, 1), 0)
    p = jnp.bitwise_and(rcol, _N - 1)
    hr = jnp.right_shift(p, 4).astype(jnp.float32) * (1.0 / (_H - 1))
    wc = jnp.bitwise_and(p, _W - 1).astype(jnp.float32) * (1.0 / (_W - 1))
    nodes_ref[...] = jnp.concatenate([x_ref[...], hr, wc], axis=1)
    batch_ref[...] = jnp.right_shift(rcol, 8)


def kernel(x, y):
    x2d = x.reshape(_R, _C)
    edges3, nodes, batch2 = pl.pallas_call(
        _build_kernel,
        grid=(_GRID,),
        in_specs=[pl.BlockSpec((_ROWS, _C), lambda g: (g, 0))],
        out_specs=[
            pl.BlockSpec((2, _ROWS, _E), lambda g: (0, g, 0)),
            pl.BlockSpec((_ROWS, _C + 2), lambda g: (g, 0)),
            pl.BlockSpec((_ROWS, 1), lambda g: (g, 0)),
        ],
        out_shape=[
            jax.ShapeDtypeStruct((2, _R, _E), jnp.int32),
            jax.ShapeDtypeStruct((_R, _C + 2), jnp.float32),
            jax.ShapeDtypeStruct((_R, 1), jnp.int32),
        ],
    )(x2d)
    edge_index = edges3.reshape(2, _R * _E)
    batch_vec = batch2.reshape(_R)
    y_out = y.reshape(_B, -1)
    return nodes, edge_index, batch_vec, y_out


# speedup vs baseline: 2.1261x; 1.9146x over previous
import jax
import jax.numpy as jnp
from jax.experimental import pallas as pl
from jax.experimental.pallas import tpu as pltpu

_B, _H, _W, _C = 32, 16, 16, 64
_N = _H * _W
_R = _B * _N
_E = _N * (_N - 1)


def _build_kernel(x_ref, edges_ref, nodes_ref, batch_ref, tmpl_ref):
    b = pl.program_id(0)

    @pl.when(b == 0)
    def _init_template():
        k = jax.lax.broadcasted_iota(jnp.int32, (1, _E), 1)
        i = jnp.right_shift(k + jnp.right_shift(k, 8) + 1, 8)
        j = k - ((i << 8) - i)
        tmpl_ref[0:1] = i
        tmpl_ref[1:2] = j + (j >= i).astype(jnp.int32)

    edges_ref[...] = tmpl_ref[...] + b * _N

    rows = jax.lax.broadcasted_iota(jnp.int32, (_N, 1), 0)
    hr = jnp.right_shift(rows, 4).astype(jnp.float32) * (1.0 / (_H - 1))
    wc = jnp.bitwise_and(rows, _W - 1).astype(jnp.float32) * (1.0 / (_W - 1))
    nodes_ref[...] = jnp.concatenate([x_ref[...], hr, wc], axis=1)
    batch_ref[...] = jnp.full((_N, 1), b, dtype=jnp.int32)


def kernel(x, y):
    x2d = x.reshape(_R, _C)
    edge_index, nodes, batch2 = pl.pallas_call(
        _build_kernel,
        grid=(_B,),
        in_specs=[pl.BlockSpec((_N, _C), lambda b: (b, 0))],
        out_specs=[
            pl.BlockSpec((2, _E), lambda b: (0, b)),
            pl.BlockSpec((_N, _C + 2), lambda b: (b, 0)),
            pl.BlockSpec((_N, 1), lambda b: (b, 0)),
        ],
        out_shape=[
            jax.ShapeDtypeStruct((2, _B * _E), jnp.int32),
            jax.ShapeDtypeStruct((_R, _C + 2), jnp.float32),
            jax.ShapeDtypeStruct((_R, 1), jnp.int32),
        ],
        scratch_shapes=[pltpu.VMEM((2, _E), jnp.int32)],
    )(x2d)
    batch_vec = batch2.reshape(_R)
    y_out = y.reshape(_B, -1)
    return nodes, edge_index, batch_vec, y_out
